# Initial kernel scaffold; baseline (speedup 1.0000x reference)
import jax
import jax.numpy as jnp
from jax.experimental import pallas as pl


def kernel(edge_idx, node_feats, edge_feats, W1e, b1e, W2e, b2e, ge, be, W1n, b1n, W2n, b2n, gn, bn):
    raise NotImplementedError("write your pallas kernel here")


# speedup vs baseline: 2.5279x; 2.5279x over previous
import functools

import jax
import jax.numpy as jnp
from jax import lax
from jax.experimental import pallas as pl
from jax.experimental.pallas import tpu as pltpu
from jax.experimental.pallas import tpu_sc as plsc

H = 128
N_NODES = 10000
N_EDGES = 320000

NC = 2
NS = 16
NW = NC * NS
EPT = N_EDGES // NW
CH = 80
NCH = EPT // CH
ACC_R = 10240
RPT = ACC_R // NS


def _sc_body(snd_hbm, rcv_hbm, ta_hbm, tb_hbm, ef_hbm, zeros_hbm,
             g_hbm, acc_hbm,
             idx_s, idx_r, rows_a, rows_b, ef_v, acc_sh,
             sem_a, sem_b, sem_e):
    cid = lax.axis_index("c")
    sid = lax.axis_index("s")
    wid = cid * NS + sid
    base = wid * EPT
    arow = pl.multiple_of(sid * RPT, 8)

    pltpu.sync_copy(zeros_hbm.at[pl.ds(arow, RPT)],
                    acc_sh.at[pl.ds(arow, RPT)])
    plsc.subcore_barrier()

    def chunk(i, carry):
        off = pl.multiple_of(base + i * CH, 8)
        pltpu.sync_copy(snd_hbm.at[pl.ds(off, CH)], idx_s)
        pltpu.sync_copy(rcv_hbm.at[pl.ds(off, CH)], idx_r)
        cp_a = pltpu.async_copy(ta_hbm.at[idx_s], rows_a, sem_a)
        cp_b = pltpu.async_copy(tb_hbm.at[idx_r], rows_b, sem_b)
        cp_e = pltpu.async_copy(ef_hbm.at[pl.ds(off, CH)], ef_v, sem_e)
        cp_a.wait()
        cp_b.wait()

        def add_row(r, c):
            for j in range(H // 16):
                sl = pl.ds(j * 16, 16)
                rows_a[r, sl] = rows_a[r, sl] + rows_b[r, sl]
            return c

        lax.fori_loop(0, CH, add_row, 0)
        pltpu.sync_copy(rows_a, g_hbm.at[pl.ds(off, CH)])
        cp_e.wait()
        pltpu.sync_copy(ef_v, acc_sh.at[idx_r], add=True)
        return carry

    lax.fori_loop(0, NCH, chunk, 0)

    plsc.subcore_barrier()
    pltpu.sync_copy(acc_sh.at[pl.ds(arow, RPT)],
                    acc_hbm.at[cid, pl.ds(arow, RPT)])


_sc_gather_scatter = pl.kernel(
    _sc_body,
    mesh=plsc.VectorSubcoreMesh(core_axis_name="c", subcore_axis_name="s"),
    out_type=[
        jax.ShapeDtypeStruct((N_EDGES, H), jnp.float32),
        jax.ShapeDtypeStruct((NC, ACC_R, H), jnp.float32),
    ],
    scratch_types=[
        pltpu.VMEM((CH,), jnp.int32),
        pltpu.VMEM((CH,), jnp.int32),
        pltpu.VMEM((CH, H), jnp.float32),
        pltpu.VMEM((CH, H), jnp.float32),
        pltpu.VMEM((CH, H), jnp.float32),
        pltpu.VMEM_SHARED((ACC_R, H), jnp.float32),
        pltpu.SemaphoreType.DMA,
        pltpu.SemaphoreType.DMA,
        pltpu.SemaphoreType.DMA,
    ],
)


def _pre_body(nf_ref, wa_ref, wb_ref, a_ref, b_ref):
    x = nf_ref[...]
    a_ref[...] = jnp.dot(x, wa_ref[...], preferred_element_type=jnp.float32)
    b_ref[...] = jnp.dot(x, wb_ref[...], preferred_element_type=jnp.float32)


def _layer_norm(x, g, b):
    m = jnp.mean(x, axis=-1, keepdims=True)
    v = jnp.mean((x - m) * (x - m), axis=-1, keepdims=True)
    return (x - m) * lax.rsqrt(v + 1e-5) * g + b


def _edge_body(g_ref, ef_ref, w1_ref, b1_ref, w2_ref, b2_ref, ge_ref, be_ref,
               out_ref):
    ef = ef_ref[...]
    h = g_ref[...] + jnp.dot(ef, w1_ref[...], preferred_element_type=jnp.float32)
    h = jnp.maximum(h + b1_ref[...], 0.0)
    eu = jnp.dot(h, w2_ref[...], preferred_element_type=jnp.float32) + b2_ref[...]
    out_ref[...] = ef + _layer_norm(eu, ge_ref[...], be_ref[...])


def _node_body(nf_ref, acc_ref, wa_ref, wb_ref, b1_ref, w2_ref, b2_ref,
               gn_ref, bn_ref, out_ref):
    nf = nf_ref[...]
    acc = (acc_ref[0] + acc_ref[1])[:N_NODES]
    h = (jnp.dot(nf, wa_ref[...], preferred_element_type=jnp.float32)
         + jnp.dot(acc, wb_ref[...], preferred_element_type=jnp.float32))
    h = jnp.maximum(h + b1_ref[...], 0.0)
    nu = jnp.dot(h, w2_ref[...], preferred_element_type=jnp.float32) + b2_ref[...]
    out_ref[...] = nf + _layer_norm(nu, gn_ref[...], bn_ref[...])


RE = 4000


def kernel(edge_idx, node_feats, edge_feats, W1e, b1e, W2e, b2e, ge, be,
           W1n, b1n, W2n, b2n, gn, bn):
    senders = edge_idx[:, 0]
    receivers = edge_idx[:, 1]
    zeros = jnp.zeros((ACC_R, H), jnp.float32)

    tab_a, tab_b = pl.pallas_call(
        _pre_body,
        out_shape=[jax.ShapeDtypeStruct((N_NODES, H), jnp.float32)] * 2,
    )(node_feats, W1e[:H], W1e[H:2 * H])

    g, acc2 = _sc_gather_scatter(senders, receivers, tab_a, tab_b,
                                 edge_feats, zeros)

    row = lambda v: v.reshape(1, H)
    edge_out = pl.pallas_call(
        _edge_body,
        grid=(N_EDGES // RE,),
        in_specs=[
            pl.BlockSpec((RE, H), lambda i: (i, 0)),
            pl.BlockSpec((RE, H), lambda i: (i, 0)),
            pl.BlockSpec((H, H), lambda i: (0, 0)),
            pl.BlockSpec((1, H), lambda i: (0, 0)),
            pl.BlockSpec((H, H), lambda i: (0, 0)),
            pl.BlockSpec((1, H), lambda i: (0, 0)),
            pl.BlockSpec((1, H), lambda i: (0, 0)),
            pl.BlockSpec((1, H), lambda i: (0, 0)),
        ],
        out_specs=pl.BlockSpec((RE, H), lambda i: (i, 0)),
        out_shape=jax.ShapeDtypeStruct((N_EDGES, H), jnp.float32),
    )(g, edge_feats, W1e[2 * H:], row(b1e), W2e, row(b2e), row(ge), row(be))

    node_out = pl.pallas_call(
        _node_body,
        out_shape=jax.ShapeDtypeStruct((N_NODES, H), jnp.float32),
    )(node_feats, acc2, W1n[:H], W1n[H:], row(b1n), W2n, row(b2n),
      row(gn), row(bn))

    return (node_out, edge_out)


# speedup vs baseline: 4.1191x; 1.6295x over previous
import jax
import jax.numpy as jnp
from jax import lax
from jax.experimental import pallas as pl
from jax.experimental.pallas import tpu as pltpu
from jax.experimental.pallas import tpu_sc as plsc

H = 128
N_NODES = 10000
N_EDGES = 320000

NC = 2
NS = 16
NW = NC * NS
EPT = N_EDGES // NW
CH = 80
NCH = EPT // CH
ACC_R = 10240
RPT = ACC_R // NS


def _scg_body(snd_hbm, rcv_hbm, ta_hbm, tb_hbm, g_hbm,
              idx_s, idx_r, rows_a, rows_b, sem_a, sem_b, sem_w, sem_is,
              sem_ir):
    cid = lax.axis_index("c")
    sid = lax.axis_index("s")
    base = (cid * NS + sid) * EPT

    def issue_idx(i, p):
        off = pl.multiple_of(base + i * CH, 16)
        pltpu.async_copy(snd_hbm.at[pl.ds(off, CH)], idx_s.at[p], sem_is.at[p])
        pltpu.async_copy(rcv_hbm.at[pl.ds(off, CH)], idx_r.at[p], sem_ir.at[p])

    def wait_idx(p):
        pltpu.make_async_copy(snd_hbm.at[pl.ds(0, CH)], idx_s.at[p],
                              sem_is.at[p]).wait()
        pltpu.make_async_copy(rcv_hbm.at[pl.ds(0, CH)], idx_r.at[p],
                              sem_ir.at[p]).wait()

    def issue(i, p, b):
        pltpu.async_copy(ta_hbm.at[idx_s.at[p]], rows_a.at[b], sem_a.at[b])
        pltpu.async_copy(tb_hbm.at[idx_r.at[p]], rows_b.at[b], sem_b.at[b])

    def wait_in(b):
        pltpu.make_async_copy(ta_hbm.at[pl.ds(0, CH)], rows_a.at[b],
                              sem_a.at[b]).wait()
        pltpu.make_async_copy(tb_hbm.at[pl.ds(0, CH)], rows_b.at[b],
                              sem_b.at[b]).wait()

    def wait_w(b):
        pltpu.make_async_copy(rows_a.at[b], g_hbm.at[pl.ds(0, CH)],
                              sem_w.at[b]).wait()

    def add_rows(b):
        def add_row(r, c):
            for j in range(H // 16):
                sl = pl.ds(j * 16, 16)
                rows_a[b, r, sl] = rows_a[b, r, sl] + rows_b[b, r, sl]
            return c

        lax.fori_loop(0, CH, add_row, 0)

    issue_idx(0, 0)
    issue_idx(1, 1)
    wait_idx(0)
    issue(0, 0, 0)

    def step(s, carry):
        for b in range(2):
            i = s * 2 + b
            nb = 1 - b
            p1 = jnp.bitwise_and(i + 1, 3)
            p2 = jnp.bitwise_and(i + 2, 3)

            wait_in(b)

            @pl.when(i + 2 < NCH)
            def _():
                issue_idx(i + 2, p2)

            @pl.when(i + 1 < NCH)
            def _():
                wait_idx(p1)

                @pl.when(i >= 1)
                def _():
                    wait_w(nb)
                issue(i + 1, p1, nb)

            add_rows(b)
            off = pl.multiple_of(base + i * CH, 16)
            pltpu.async_copy(rows_a.at[b], g_hbm.at[pl.ds(off, CH)],
                             sem_w.at[b])
        return carry

    lax.fori_loop(0, NCH // 2, step, 0)

    wait_in(0)
    add_rows(0)
    pltpu.sync_copy(rows_a.at[0],
                    g_hbm.at[pl.ds(pl.multiple_of(base + (NCH - 1) * CH, 16),
                                   CH)])
    wait_w(1)


_sc_gather = pl.kernel(
    _scg_body,
    mesh=plsc.VectorSubcoreMesh(core_axis_name="c", subcore_axis_name="s"),
    out_type=[jax.ShapeDtypeStruct((N_EDGES, H), jnp.float32)],
    scratch_types=[
        pltpu.VMEM((4, CH), jnp.int32),
        pltpu.VMEM((4, CH), jnp.int32),
        pltpu.VMEM((2, CH, H), jnp.float32),
        pltpu.VMEM((2, CH, H), jnp.float32),
        pltpu.SemaphoreType.DMA((2,)),
        pltpu.SemaphoreType.DMA((2,)),
        pltpu.SemaphoreType.DMA((2,)),
        pltpu.SemaphoreType.DMA((4,)),
        pltpu.SemaphoreType.DMA((4,)),
    ],
)


def _scs_body(rcv_hbm, ef_hbm, zeros_hbm, acc_hbm,
              idx_r, ef_v, acc_sh, sem_e, sem_ir):
    cid = lax.axis_index("c")
    sid = lax.axis_index("s")
    base = (cid * NS + sid) * EPT
    arow = pl.multiple_of(sid * RPT, 8)

    def issue_idx(i, p):
        off = pl.multiple_of(base + i * CH, 16)
        pltpu.async_copy(rcv_hbm.at[pl.ds(off, CH)], idx_r.at[p], sem_ir.at[p])

    def wait_idx(p):
        pltpu.make_async_copy(rcv_hbm.at[pl.ds(0, CH)], idx_r.at[p],
                              sem_ir.at[p]).wait()

    def issue_ef(i, b):
        off = pl.multiple_of(base + i * CH, 16)
        pltpu.async_copy(ef_hbm.at[pl.ds(off, CH)], ef_v.at[b], sem_e.at[b])

    def wait_ef(b):
        pltpu.make_async_copy(ef_hbm.at[pl.ds(0, CH)], ef_v.at[b],
                              sem_e.at[b]).wait()

    issue_idx(0, 0)
    issue_idx(1, 1)
    issue_ef(0, 0)
    pltpu.sync_copy(zeros_hbm.at[pl.ds(arow, RPT)],
                    acc_sh.at[pl.ds(arow, RPT)])
    plsc.subcore_barrier()

    def step(s, carry):
        for b in range(2):
            i = s * 2 + b
            nb = 1 - b
            p = jnp.bitwise_and(i, 3)
            p2 = jnp.bitwise_and(i + 2, 3)

            wait_ef(b)
            wait_idx(p)

            @pl.when(i + 1 < NCH)
            def _():
                issue_ef(i + 1, nb)

            @pl.when(i + 2 < NCH)
            def _():
                issue_idx(i + 2, p2)

            pltpu.sync_copy(ef_v.at[b], acc_sh.at[idx_r.at[p]], add=True)
        return carry

    lax.fori_loop(0, NCH // 2, step, 0)

    last = NCH - 1
    wait_ef(0)
    wait_idx(last % 4)
    pltpu.sync_copy(ef_v.at[0], acc_sh.at[idx_r.at[last % 4]], add=True)

    plsc.subcore_barrier()
    pltpu.sync_copy(acc_sh.at[pl.ds(arow, RPT)],
                    acc_hbm.at[cid, pl.ds(arow, RPT)])


_sc_scatter = pl.kernel(
    _scs_body,
    mesh=plsc.VectorSubcoreMesh(core_axis_name="c", subcore_axis_name="s"),
    out_type=[jax.ShapeDtypeStruct((NC, ACC_R, H), jnp.float32)],
    scratch_types=[
        pltpu.VMEM((4, CH), jnp.int32),
        pltpu.VMEM((2, CH, H), jnp.float32),
        pltpu.VMEM_SHARED((ACC_R, H), jnp.float32),
        pltpu.SemaphoreType.DMA((2,)),
        pltpu.SemaphoreType.DMA((4,)),
    ],
)


def _pre_body(nf_ref, wa_ref, wb_ref, a_ref, b_ref):
    x = nf_ref[...]
    a_ref[...] = jnp.dot(x, wa_ref[...], preferred_element_type=jnp.float32)
    b_ref[...] = jnp.dot(x, wb_ref[...], preferred_element_type=jnp.float32)


def _layer_norm(x, g, b):
    m = jnp.mean(x, axis=-1, keepdims=True)
    v = jnp.mean((x - m) * (x - m), axis=-1, keepdims=True)
    return (x - m) * lax.rsqrt(v + 1e-5) * g + b


def _edge_body(g_ref, ef_ref, w1_ref, b1_ref, w2_ref, b2_ref, ge_ref, be_ref,
               out_ref):
    ef = ef_ref[...]
    h = (g_ref[...]
         + jnp.dot(ef, w1_ref[...], preferred_element_type=jnp.float32))
    h = jnp.maximum(h + b1_ref[...], 0.0)
    eu = jnp.dot(h, w2_ref[...], preferred_element_type=jnp.float32) + b2_ref[...]
    out_ref[...] = ef + _layer_norm(eu, ge_ref[...], be_ref[...])


def _node_body(nf_ref, acc_ref, wa_ref, wb_ref, b1_ref, w2_ref, b2_ref,
               gn_ref, bn_ref, out_ref):
    nf = nf_ref[...]
    acc = (acc_ref[0] + acc_ref[1])[:N_NODES]
    h = (jnp.dot(nf, wa_ref[...], preferred_element_type=jnp.float32)
         + jnp.dot(acc, wb_ref[...], preferred_element_type=jnp.float32))
    h = jnp.maximum(h + b1_ref[...], 0.0)
    nu = jnp.dot(h, w2_ref[...], preferred_element_type=jnp.float32) + b2_ref[...]
    out_ref[...] = nf + _layer_norm(nu, gn_ref[...], bn_ref[...])


RE = 4000


def kernel(edge_idx, node_feats, edge_feats, W1e, b1e, W2e, b2e, ge, be,
           W1n, b1n, W2n, b2n, gn, bn):
    senders = edge_idx[:, 0]
    receivers = edge_idx[:, 1]
    zeros = jnp.zeros((ACC_R, H), jnp.float32)

    tab_a, tab_b = pl.pallas_call(
        _pre_body,
        out_shape=[jax.ShapeDtypeStruct((N_NODES, H), jnp.float32)] * 2,
    )(node_feats, W1e[:H], W1e[H:2 * H])

    (g,) = _sc_gather(senders, receivers, tab_a, tab_b)
    (acc2,) = _sc_scatter(receivers, edge_feats, zeros)

    row = lambda v: v.reshape(1, H)
    edge_out = pl.pallas_call(
        _edge_body,
        grid=(N_EDGES // RE,),
        in_specs=[
            pl.BlockSpec((RE, H), lambda i: (i, 0)),
            pl.BlockSpec((RE, H), lambda i: (i, 0)),
            pl.BlockSpec((H, H), lambda i: (0, 0)),
            pl.BlockSpec((1, H), lambda i: (0, 0)),
            pl.BlockSpec((H, H), lambda i: (0, 0)),
            pl.BlockSpec((1, H), lambda i: (0, 0)),
            pl.BlockSpec((1, H), lambda i: (0, 0)),
            pl.BlockSpec((1, H), lambda i: (0, 0)),
        ],
        out_specs=pl.BlockSpec((RE, H), lambda i: (i, 0)),
        out_shape=jax.ShapeDtypeStruct((N_EDGES, H), jnp.float32),
    )(g, edge_feats, W1e[2 * H:], row(b1e), W2e, row(b2e), row(ge), row(be))

    node_out = pl.pallas_call(
        _node_body,
        out_shape=jax.ShapeDtypeStruct((N_NODES, H), jnp.float32),
    )(node_feats, acc2, W1n[:H], W1n[H:], row(b1n), W2n, row(b2n),
      row(gn), row(bn))

    return (node_out, edge_out)


# speedup vs baseline: 4.1221x; 1.0007x over previous
import jax
import jax.numpy as jnp
from jax import lax
from jax.experimental import pallas as pl
from jax.experimental.pallas import tpu as pltpu
from jax.experimental.pallas import tpu_sc as plsc

H = 128
N_NODES = 10000
N_EDGES = 320000

NC = 2
NS = 16
NW = NC * NS
EPT = N_EDGES // NW
CH = 80
NCH = EPT // CH
ACC_R = 10240
RPT = ACC_R // NS


def _scg_body(snd_hbm, rcv_hbm, ta_hbm, tb_hbm, g_hbm,
              idx_s, idx_r, rows_a, rows_b, sem_a, sem_b, sem_w, sem_is,
              sem_ir):
    cid = lax.axis_index("c")
    sid = lax.axis_index("s")
    base = (cid * NS + sid) * EPT

    def issue_idx(i, p):
        off = pl.multiple_of(base + i * CH, 16)
        pltpu.async_copy(snd_hbm.at[pl.ds(off, CH)], idx_s.at[p], sem_is.at[p])
        pltpu.async_copy(rcv_hbm.at[pl.ds(off, CH)], idx_r.at[p], sem_ir.at[p])

    def wait_idx(p):
        pltpu.make_async_copy(snd_hbm.at[pl.ds(0, CH)], idx_s.at[p],
                              sem_is.at[p]).wait()
        pltpu.make_async_copy(rcv_hbm.at[pl.ds(0, CH)], idx_r.at[p],
                              sem_ir.at[p]).wait()

    def issue(i, p, b):
        pltpu.async_copy(ta_hbm.at[idx_s.at[p]], rows_a.at[b], sem_a.at[b])
        pltpu.async_copy(tb_hbm.at[idx_r.at[p]], rows_b.at[b], sem_b.at[b])

    def wait_in(b):
        pltpu.make_async_copy(ta_hbm.at[pl.ds(0, CH)], rows_a.at[b],
                              sem_a.at[b]).wait()
        pltpu.make_async_copy(tb_hbm.at[pl.ds(0, CH)], rows_b.at[b],
                              sem_b.at[b]).wait()

    def wait_w(b):
        pltpu.make_async_copy(rows_a.at[b], g_hbm.at[pl.ds(0, CH)],
                              sem_w.at[b]).wait()

    def add_rows(b):
        def add_row(r, c):
            for j in range(H // 16):
                sl = pl.ds(j * 16, 16)
                rows_a[b, r, sl] = rows_a[b, r, sl] + rows_b[b, r, sl]
            return c

        lax.fori_loop(0, CH, add_row, 0)

    issue_idx(0, 0)
    issue_idx(1, 1)
    wait_idx(0)
    issue(0, 0, 0)

    def step(s, carry):
        for b in range(2):
            i = s * 2 + b
            nb = 1 - b
            p1 = jnp.bitwise_and(i + 1, 3)
            p2 = jnp.bitwise_and(i + 2, 3)

            wait_in(b)

            @pl.when(i + 2 < NCH)
            def _():
                issue_idx(i + 2, p2)

            @pl.when(i + 1 < NCH)
            def _():
                wait_idx(p1)

                @pl.when(i >= 1)
                def _():
                    wait_w(nb)
                issue(i + 1, p1, nb)

            add_rows(b)
            off = pl.multiple_of(base + i * CH, 16)
            pltpu.async_copy(rows_a.at[b], g_hbm.at[pl.ds(off, CH)],
                             sem_w.at[b])
        return carry

    lax.fori_loop(0, NCH // 2, step, 0)

    wait_in(0)
    add_rows(0)
    pltpu.sync_copy(rows_a.at[0],
                    g_hbm.at[pl.ds(pl.multiple_of(base + (NCH - 1) * CH, 16),
                                   CH)])
    wait_w(1)


_sc_gather = pl.kernel(
    _scg_body,
    mesh=plsc.VectorSubcoreMesh(core_axis_name="c", subcore_axis_name="s"),
    out_type=[jax.ShapeDtypeStruct((N_EDGES, H), jnp.float32)],
    scratch_types=[
        pltpu.VMEM((4, CH), jnp.int32),
        pltpu.VMEM((4, CH), jnp.int32),
        pltpu.VMEM((2, CH, H), jnp.float32),
        pltpu.VMEM((2, CH, H), jnp.float32),
        pltpu.SemaphoreType.DMA((2,)),
        pltpu.SemaphoreType.DMA((2,)),
        pltpu.SemaphoreType.DMA((2,)),
        pltpu.SemaphoreType.DMA((4,)),
        pltpu.SemaphoreType.DMA((4,)),
    ],
)


def _scs_body(rcv_hbm, ef_hbm, zeros_hbm, acc_hbm,
              idx_r, ef_v, acc_sh, sem_e, sem_ir, sem_sc):
    cid = lax.axis_index("c")
    sid = lax.axis_index("s")
    base = (cid * NS + sid) * EPT
    arow = pl.multiple_of(sid * RPT, 8)

    def issue_idx(i, p):
        off = pl.multiple_of(base + i * CH, 16)
        pltpu.async_copy(rcv_hbm.at[pl.ds(off, CH)], idx_r.at[p], sem_ir.at[p])

    def wait_idx(p):
        pltpu.make_async_copy(rcv_hbm.at[pl.ds(0, CH)], idx_r.at[p],
                              sem_ir.at[p]).wait()

    def issue_ef(i, b):
        off = pl.multiple_of(base + i * CH, 16)
        pltpu.async_copy(ef_hbm.at[pl.ds(off, CH)], ef_v.at[b], sem_e.at[b])

    def wait_ef(b):
        pltpu.make_async_copy(ef_hbm.at[pl.ds(0, CH)], ef_v.at[b],
                              sem_e.at[b]).wait()

    def wait_sc(b):
        pltpu.make_async_copy(ef_hbm.at[pl.ds(0, CH)], ef_v.at[b],
                              sem_sc.at[b]).wait()

    issue_idx(0, 0)
    issue_idx(1, 1)
    issue_ef(0, 0)
    pltpu.sync_copy(zeros_hbm.at[pl.ds(arow, RPT)],
                    acc_sh.at[pl.ds(arow, RPT)])
    plsc.subcore_barrier()

    def step(s, carry):
        for b in range(2):
            i = s * 2 + b
            nb = 1 - b
            p = jnp.bitwise_and(i, 3)
            p2 = jnp.bitwise_and(i + 2, 3)

            wait_ef(b)
            wait_idx(p)
            pltpu.async_copy(ef_v.at[b], acc_sh.at[idx_r.at[p]],
                             sem_sc.at[b], add=True)

            @pl.when(i + 1 < NCH)
            def _():
                @pl.when(i >= 1)
                def _():
                    wait_sc(nb)
                issue_ef(i + 1, nb)

            @pl.when(i + 2 < NCH)
            def _():
                issue_idx(i + 2, p2)
        return carry

    lax.fori_loop(0, NCH // 2, step, 0)

    last = NCH - 1
    wait_ef(0)
    wait_idx(last % 4)
    pltpu.sync_copy(ef_v.at[0], acc_sh.at[idx_r.at[last % 4]], add=True)
    wait_sc(1)

    plsc.subcore_barrier()
    pltpu.sync_copy(acc_sh.at[pl.ds(arow, RPT)],
                    acc_hbm.at[cid, pl.ds(arow, RPT)])


_sc_scatter = pl.kernel(
    _scs_body,
    mesh=plsc.VectorSubcoreMesh(core_axis_name="c", subcore_axis_name="s"),
    out_type=[jax.ShapeDtypeStruct((NC, ACC_R, H), jnp.float32)],
    scratch_types=[
        pltpu.VMEM((4, CH), jnp.int32),
        pltpu.VMEM((2, CH, H), jnp.float32),
        pltpu.VMEM_SHARED((ACC_R, H), jnp.float32),
        pltpu.SemaphoreType.DMA((2,)),
        pltpu.SemaphoreType.DMA((4,)),
        pltpu.SemaphoreType.DMA((2,)),
    ],
)


def _pre_body(nf_ref, wa_ref, wb_ref, a_ref, b_ref):
    x = nf_ref[...]
    a_ref[...] = jnp.dot(x, wa_ref[...], preferred_element_type=jnp.float32)
    b_ref[...] = jnp.dot(x, wb_ref[...], preferred_element_type=jnp.float32)


def _layer_norm(x, g, b):
    m = jnp.mean(x, axis=-1, keepdims=True)
    v = jnp.mean((x - m) * (x - m), axis=-1, keepdims=True)
    return (x - m) * lax.rsqrt(v + 1e-5) * g + b


def _edge_body(g_ref, ef_ref, w1_ref, b1_ref, w2_ref, b2_ref, ge_ref, be_ref,
               out_ref):
    ef = ef_ref[...]
    h = (g_ref[...]
         + jnp.dot(ef, w1_ref[...], preferred_element_type=jnp.float32))
    h = jnp.maximum(h + b1_ref[...], 0.0)
    eu = jnp.dot(h, w2_ref[...], preferred_element_type=jnp.float32) + b2_ref[...]
    out_ref[...] = ef + _layer_norm(eu, ge_ref[...], be_ref[...])


def _node_body(nf_ref, acc_ref, wa_ref, wb_ref, b1_ref, w2_ref, b2_ref,
               gn_ref, bn_ref, out_ref):
    nf = nf_ref[...]
    acc = (acc_ref[0] + acc_ref[1])[:N_NODES]
    h = (jnp.dot(nf, wa_ref[...], preferred_element_type=jnp.float32)
         + jnp.dot(acc, wb_ref[...], preferred_element_type=jnp.float32))
    h = jnp.maximum(h + b1_ref[...], 0.0)
    nu = jnp.dot(h, w2_ref[...], preferred_element_type=jnp.float32) + b2_ref[...]
    out_ref[...] = nf + _layer_norm(nu, gn_ref[...], bn_ref[...])


RE = 4000


def kernel(edge_idx, node_feats, edge_feats, W1e, b1e, W2e, b2e, ge, be,
           W1n, b1n, W2n, b2n, gn, bn):
    senders = edge_idx[:, 0]
    receivers = edge_idx[:, 1]
    zeros = jnp.zeros((ACC_R, H), jnp.float32)

    tab_a, tab_b = pl.pallas_call(
        _pre_body,
        out_shape=[jax.ShapeDtypeStruct((N_NODES, H), jnp.float32)] * 2,
    )(node_feats, W1e[:H], W1e[H:2 * H])

    (g,) = _sc_gather(senders, receivers, tab_a, tab_b)
    (acc2,) = _sc_scatter(receivers, edge_feats, zeros)

    row = lambda v: v.reshape(1, H)
    edge_out = pl.pallas_call(
        _edge_body,
        grid=(N_EDGES // RE,),
        in_specs=[
            pl.BlockSpec((RE, H), lambda i: (i, 0)),
            pl.BlockSpec((RE, H), lambda i: (i, 0)),
            pl.BlockSpec((H, H), lambda i: (0, 0)),
            pl.BlockSpec((1, H), lambda i: (0, 0)),
            pl.BlockSpec((H, H), lambda i: (0, 0)),
            pl.BlockSpec((1, H), lambda i: (0, 0)),
            pl.BlockSpec((1, H), lambda i: (0, 0)),
            pl.BlockSpec((1, H), lambda i: (0, 0)),
        ],
        out_specs=pl.BlockSpec((RE, H), lambda i: (i, 0)),
        out_shape=jax.ShapeDtypeStruct((N_EDGES, H), jnp.float32),
    )(g, edge_feats, W1e[2 * H:], row(b1e), W2e, row(b2e), row(ge), row(be))

    node_out = pl.pallas_call(
        _node_body,
        out_shape=jax.ShapeDtypeStruct((N_NODES, H), jnp.float32),
    )(node_feats, acc2, W1n[:H], W1n[H:], row(b1n), W2n, row(b2n),
      row(gn), row(bn))

    return (node_out, edge_out)


# speedup vs baseline: 4.1345x; 1.0030x over previous
import jax
import jax.numpy as jnp
from jax import lax
from jax.experimental import pallas as pl
from jax.experimental.pallas import tpu as pltpu
from jax.experimental.pallas import tpu_sc as plsc

H = 128
HP = H // 2
N_NODES = 10000
N_EDGES = 320000

NC = 2
NS = 16
NW = NC * NS
EPT = N_EDGES // NW
CH = 80
NCH = EPT // CH
ACC_R = 10240
RPT = ACC_R // NS


def _scg_body(snd_hbm, rcv_hbm, ta_hbm, tb_hbm, g_hbm,
              idx_s, idx_r, rows_a, rows_b, rows_g, sem_a, sem_b, sem_w,
              sem_is, sem_ir):
    cid = lax.axis_index("c")
    sid = lax.axis_index("s")
    base = (cid * NS + sid) * EPT

    def issue_idx(i, p):
        off = pl.multiple_of(base + i * CH, 16)
        pltpu.async_copy(snd_hbm.at[pl.ds(off, CH)], idx_s.at[p], sem_is.at[p])
        pltpu.async_copy(rcv_hbm.at[pl.ds(off, CH)], idx_r.at[p], sem_ir.at[p])

    def wait_idx(p):
        pltpu.make_async_copy(snd_hbm.at[pl.ds(0, CH)], idx_s.at[p],
                              sem_is.at[p]).wait()
        pltpu.make_async_copy(rcv_hbm.at[pl.ds(0, CH)], idx_r.at[p],
                              sem_ir.at[p]).wait()

    def issue(i, p, b):
        pltpu.async_copy(ta_hbm.at[idx_s.at[p]], rows_a.at[b], sem_a.at[b])
        pltpu.async_copy(tb_hbm.at[idx_r.at[p]], rows_b.at[b], sem_b.at[b])

    def wait_in(b):
        pltpu.make_async_copy(ta_hbm.at[pl.ds(0, CH)], rows_a.at[b],
                              sem_a.at[b]).wait()
        pltpu.make_async_copy(tb_hbm.at[pl.ds(0, CH)], rows_b.at[b],
                              sem_b.at[b]).wait()

    def wait_w(b):
        pltpu.make_async_copy(rows_g.at[b], g_hbm.at[pl.ds(0, CH)],
                              sem_w.at[b]).wait()

    def add_rows(b):
        half = jnp.int32(0x8000)
        mask = jnp.int32(-65536)

        def add_row(r, c):
            for j in range(HP // 16):
                lo = (rows_a[b, r, pl.ds(j * 16, 16)]
                      + rows_b[b, r, pl.ds(j * 16, 16)])
                hi = (rows_a[b, r, pl.ds(j * 16 + 64, 16)]
                      + rows_b[b, r, pl.ds(j * 16 + 64, 16)])
                lo_i = lax.shift_right_logical(
                    lax.bitcast_convert_type(lo, jnp.int32) + half, 16)
                hi_i = (lax.bitcast_convert_type(hi, jnp.int32) + half) & mask
                rows_g[b, r, pl.ds(j * 16, 16)] = lo_i | hi_i
            return c

        lax.fori_loop(0, CH, add_row, 0)

    issue_idx(0, 0)
    issue_idx(1, 1)
    wait_idx(0)
    issue(0, 0, 0)

    def step(s, carry):
        for b in range(2):
            i = s * 2 + b
            nb = 1 - b
            p1 = jnp.bitwise_and(i + 1, 3)
            p2 = jnp.bitwise_and(i + 2, 3)

            wait_in(b)

            @pl.when(i + 2 < NCH)
            def _():
                issue_idx(i + 2, p2)

            @pl.when(i + 1 < NCH)
            def _():
                wait_idx(p1)

                @pl.when(i >= 1)
                def _():
                    wait_w(nb)
                issue(i + 1, p1, nb)

            add_rows(b)
            off = pl.multiple_of(base + i * CH, 16)
            pltpu.async_copy(rows_g.at[b], g_hbm.at[pl.ds(off, CH)],
                             sem_w.at[b])
        return carry

    lax.fori_loop(0, NCH // 2, step, 0)

    wait_in(0)
    add_rows(0)
    pltpu.sync_copy(rows_g.at[0],
                    g_hbm.at[pl.ds(pl.multiple_of(base + (NCH - 1) * CH, 16),
                                   CH)])
    wait_w(1)


_sc_gather = pl.kernel(
    _scg_body,
    mesh=plsc.VectorSubcoreMesh(core_axis_name="c", subcore_axis_name="s"),
    out_type=[jax.ShapeDtypeStruct((N_EDGES, HP), jnp.int32)],
    scratch_types=[
        pltpu.VMEM((4, CH), jnp.int32),
        pltpu.VMEM((4, CH), jnp.int32),
        pltpu.VMEM((2, CH, H), jnp.float32),
        pltpu.VMEM((2, CH, H), jnp.float32),
        pltpu.VMEM((2, CH, HP), jnp.int32),
        pltpu.SemaphoreType.DMA((2,)),
        pltpu.SemaphoreType.DMA((2,)),
        pltpu.SemaphoreType.DMA((2,)),
        pltpu.SemaphoreType.DMA((4,)),
        pltpu.SemaphoreType.DMA((4,)),
    ],
)


def _scs_body(rcv_hbm, ef_hbm, zeros_hbm, acc_hbm,
              idx_r, ef_v, acc_sh, sem_e, sem_ir, sem_sc):
    cid = lax.axis_index("c")
    sid = lax.axis_index("s")
    base = (cid * NS + sid) * EPT
    arow = pl.multiple_of(sid * RPT, 8)

    def issue_idx(i, p):
        off = pl.multiple_of(base + i * CH, 16)
        pltpu.async_copy(rcv_hbm.at[pl.ds(off, CH)], idx_r.at[p], sem_ir.at[p])

    def wait_idx(p):
        pltpu.make_async_copy(rcv_hbm.at[pl.ds(0, CH)], idx_r.at[p],
                              sem_ir.at[p]).wait()

    def issue_ef(i, b):
        off = pl.multiple_of(base + i * CH, 16)
        pltpu.async_copy(ef_hbm.at[pl.ds(off, CH)], ef_v.at[b], sem_e.at[b])

    def wait_ef(b):
        pltpu.make_async_copy(ef_hbm.at[pl.ds(0, CH)], ef_v.at[b],
                              sem_e.at[b]).wait()

    def wait_sc(b):
        pltpu.make_async_copy(ef_hbm.at[pl.ds(0, CH)], ef_v.at[b],
                              sem_sc.at[b]).wait()

    issue_idx(0, 0)
    issue_idx(1, 1)
    issue_ef(0, 0)
    pltpu.sync_copy(zeros_hbm.at[pl.ds(arow, RPT)],
                    acc_sh.at[pl.ds(arow, RPT)])
    plsc.subcore_barrier()

    def step(s, carry):
        for b in range(2):
            i = s * 2 + b
            nb = 1 - b
            p = jnp.bitwise_and(i, 3)
            p2 = jnp.bitwise_and(i + 2, 3)

            wait_ef(b)
            wait_idx(p)
            pltpu.async_copy(ef_v.at[b], acc_sh.at[idx_r.at[p]],
                             sem_sc.at[b], add=True)

            @pl.when(i + 1 < NCH)
            def _():
                @pl.when(i >= 1)
                def _():
                    wait_sc(nb)
                issue_ef(i + 1, nb)

            @pl.when(i + 2 < NCH)
            def _():
                issue_idx(i + 2, p2)
        return carry

    lax.fori_loop(0, NCH // 2, step, 0)

    last = NCH - 1
    wait_ef(0)
    wait_idx(last % 4)
    pltpu.sync_copy(ef_v.at[0], acc_sh.at[idx_r.at[last % 4]], add=True)
    wait_sc(1)

    plsc.subcore_barrier()
    pltpu.sync_copy(acc_sh.at[pl.ds(arow, RPT)],
                    acc_hbm.at[cid, pl.ds(arow, RPT)])


_sc_scatter = pl.kernel(
    _scs_body,
    mesh=plsc.VectorSubcoreMesh(core_axis_name="c", subcore_axis_name="s"),
    out_type=[jax.ShapeDtypeStruct((NC, ACC_R, H), jnp.float32)],
    scratch_types=[
        pltpu.VMEM((4, CH), jnp.int32),
        pltpu.VMEM((2, CH, H), jnp.float32),
        pltpu.VMEM_SHARED((ACC_R, H), jnp.float32),
        pltpu.SemaphoreType.DMA((2,)),
        pltpu.SemaphoreType.DMA((4,)),
        pltpu.SemaphoreType.DMA((2,)),
    ],
)


def _pre_body(nf_ref, wa_ref, wb_ref, a_ref, b_ref):
    x = nf_ref[...]
    a_ref[...] = jnp.dot(x, wa_ref[...], preferred_element_type=jnp.float32)
    b_ref[...] = jnp.dot(x, wb_ref[...], preferred_element_type=jnp.float32)


def _layer_norm(x, g, b):
    m = jnp.mean(x, axis=-1, keepdims=True)
    v = jnp.mean((x - m) * (x - m), axis=-1, keepdims=True)
    return (x - m) * lax.rsqrt(v + 1e-5) * g + b


def _edge_body(g_ref, ef_ref, w1_ref, b1_ref, w2_ref, b2_ref, ge_ref, be_ref,
               out_ref):
    gi = g_ref[...]
    gp = jnp.concatenate(
        [lax.bitcast_convert_type(gi << 16, jnp.float32),
         lax.bitcast_convert_type(gi & jnp.int32(-65536), jnp.float32)],
        axis=1)
    ef = ef_ref[...]
    h = (gp
         + jnp.dot(ef, w1_ref[...], preferred_element_type=jnp.float32))
    h = jnp.maximum(h + b1_ref[...], 0.0)
    eu = jnp.dot(h, w2_ref[...], preferred_element_type=jnp.float32) + b2_ref[...]
    out_ref[...] = ef + _layer_norm(eu, ge_ref[...], be_ref[...])


def _node_body(nf_ref, acc_ref, wa_ref, wb_ref, b1_ref, w2_ref, b2_ref,
               gn_ref, bn_ref, out_ref):
    nf = nf_ref[...]
    acc = (acc_ref[0] + acc_ref[1])[:N_NODES]
    h = (jnp.dot(nf, wa_ref[...], preferred_element_type=jnp.float32)
         + jnp.dot(acc, wb_ref[...], preferred_element_type=jnp.float32))
    h = jnp.maximum(h + b1_ref[...], 0.0)
    nu = jnp.dot(h, w2_ref[...], preferred_element_type=jnp.float32) + b2_ref[...]
    out_ref[...] = nf + _layer_norm(nu, gn_ref[...], bn_ref[...])


RE = 4000


def kernel(edge_idx, node_feats, edge_feats, W1e, b1e, W2e, b2e, ge, be,
           W1n, b1n, W2n, b2n, gn, bn):
    senders = edge_idx[:, 0]
    receivers = edge_idx[:, 1]
    zeros = jnp.zeros((ACC_R, H), jnp.float32)

    tab_a, tab_b = pl.pallas_call(
        _pre_body,
        out_shape=[jax.ShapeDtypeStruct((N_NODES, H), jnp.float32)] * 2,
    )(node_feats, W1e[:H], W1e[H:2 * H])

    (g,) = _sc_gather(senders, receivers, tab_a, tab_b)
    (acc2,) = _sc_scatter(receivers, edge_feats, zeros)

    row = lambda v: v.reshape(1, H)
    edge_out = pl.pallas_call(
        _edge_body,
        grid=(N_EDGES // RE,),
        in_specs=[
            pl.BlockSpec((RE, HP), lambda i: (i, 0)),
            pl.BlockSpec((RE, H), lambda i: (i, 0)),
            pl.BlockSpec((H, H), lambda i: (0, 0)),
            pl.BlockSpec((1, H), lambda i: (0, 0)),
            pl.BlockSpec((H, H), lambda i: (0, 0)),
            pl.BlockSpec((1, H), lambda i: (0, 0)),
            pl.BlockSpec((1, H), lambda i: (0, 0)),
            pl.BlockSpec((1, H), lambda i: (0, 0)),
        ],
        out_specs=pl.BlockSpec((RE, H), lambda i: (i, 0)),
        out_shape=jax.ShapeDtypeStruct((N_EDGES, H), jnp.float32),
    )(g, edge_feats, W1e[2 * H:], row(b1e), W2e, row(b2e), row(ge), row(be))

    node_out = pl.pallas_call(
        _node_body,
        out_shape=jax.ShapeDtypeStruct((N_NODES, H), jnp.float32),
    )(node_feats, acc2, W1n[:H], W1n[H:], row(b1n), W2n, row(b2n),
      row(gn), row(bn))

    return (node_out, edge_out)


# speedup vs baseline: 4.3708x; 1.0572x over previous
import jax
import jax.numpy as jnp
from jax import lax
from jax.experimental import pallas as pl
from jax.experimental.pallas import tpu as pltpu
from jax.experimental.pallas import tpu_sc as plsc

H = 128
HP = H // 2
N_NODES = 10000
N_EDGES = 320000

NC = 2
NS = 16
NW = NC * NS
EPT = N_EDGES // NW
CH = 80
NCH = EPT // CH
ACC_R = 10240
RPT = ACC_R // NS


def _scg_body(snd_hbm, rcv_hbm, ta_hbm, tb_hbm, g_hbm,
              idx_s, idx_r, rows_a, rows_b, rows_g, sem_a, sem_b, sem_w,
              sem_is, sem_ir):
    cid = lax.axis_index("c")
    sid = lax.axis_index("s")
    base = (cid * NS + sid) * EPT

    def issue_idx(i, p):
        off = pl.multiple_of(base + i * CH, 16)
        pltpu.async_copy(snd_hbm.at[pl.ds(off, CH)], idx_s.at[p], sem_is.at[p])
        pltpu.async_copy(rcv_hbm.at[pl.ds(off, CH)], idx_r.at[p], sem_ir.at[p])

    def wait_idx(p):
        pltpu.make_async_copy(snd_hbm.at[pl.ds(0, CH)], idx_s.at[p],
                              sem_is.at[p]).wait()
        pltpu.make_async_copy(rcv_hbm.at[pl.ds(0, CH)], idx_r.at[p],
                              sem_ir.at[p]).wait()

    def issue(i, p, b):
        pltpu.async_copy(ta_hbm.at[idx_s.at[p]], rows_a.at[b], sem_a.at[b])
        pltpu.async_copy(tb_hbm.at[idx_r.at[p]], rows_b.at[b], sem_b.at[b])

    def wait_in(b):
        pltpu.make_async_copy(ta_hbm.at[pl.ds(0, CH)], rows_a.at[b],
                              sem_a.at[b]).wait()
        pltpu.make_async_copy(tb_hbm.at[pl.ds(0, CH)], rows_b.at[b],
                              sem_b.at[b]).wait()

    def wait_w(b):
        pltpu.make_async_copy(rows_g.at[b], g_hbm.at[pl.ds(0, CH)],
                              sem_w.at[b]).wait()

    def add_rows(b):
        half = jnp.int32(0x8000)
        mask = jnp.int32(-65536)

        def add_row(r, c):
            for j in range(HP // 16):
                lo = (rows_a[b, r, pl.ds(j * 16, 16)]
                      + rows_b[b, r, pl.ds(j * 16, 16)])
                hi = (rows_a[b, r, pl.ds(j * 16 + 64, 16)]
                      + rows_b[b, r, pl.ds(j * 16 + 64, 16)])
                lo_i = lax.shift_right_logical(
                    lax.bitcast_convert_type(lo, jnp.int32) + half, 16)
                hi_i = (lax.bitcast_convert_type(hi, jnp.int32) + half) & mask
                rows_g[b, r, pl.ds(j * 16, 16)] = lo_i | hi_i
            return c

        lax.fori_loop(0, CH, add_row, 0)

    for k in range(6):
        issue_idx(k, k)
    for k in range(3):
        wait_idx(k)
        issue(k, k, k)

    def step(s, carry):
        for b in range(4):
            i = s * 4 + b
            b3 = (b + 3) % 4
            p3 = jnp.bitwise_and(i + 3, 7)
            p6 = jnp.bitwise_and(i + 6, 7)

            wait_in(b)

            @pl.when(i + 6 < NCH)
            def _():
                issue_idx(i + 6, p6)

            @pl.when(i + 3 < NCH)
            def _():
                wait_idx(p3)

                @pl.when(i >= 1)
                def _():
                    wait_w(b3)
                issue(i + 3, p3, b3)

            add_rows(b)
            off = pl.multiple_of(base + i * CH, 16)
            pltpu.async_copy(rows_g.at[b], g_hbm.at[pl.ds(off, CH)],
                             sem_w.at[b])
        return carry

    lax.fori_loop(0, NCH // 4, step, 0)

    wait_in(0)
    add_rows(0)
    pltpu.sync_copy(rows_g.at[0],
                    g_hbm.at[pl.ds(pl.multiple_of(base + (NCH - 1) * CH, 16),
                                   CH)])
    wait_w(1)
    wait_w(2)
    wait_w(3)


_sc_gather = pl.kernel(
    _scg_body,
    mesh=plsc.VectorSubcoreMesh(core_axis_name="c", subcore_axis_name="s"),
    out_type=[jax.ShapeDtypeStruct((N_EDGES, HP), jnp.int32)],
    scratch_types=[
        pltpu.VMEM((8, CH), jnp.int32),
        pltpu.VMEM((8, CH), jnp.int32),
        pltpu.VMEM((4, CH, H), jnp.float32),
        pltpu.VMEM((4, CH, H), jnp.float32),
        pltpu.VMEM((4, CH, HP), jnp.int32),
        pltpu.SemaphoreType.DMA((4,)),
        pltpu.SemaphoreType.DMA((4,)),
        pltpu.SemaphoreType.DMA((4,)),
        pltpu.SemaphoreType.DMA((8,)),
        pltpu.SemaphoreType.DMA((8,)),
    ],
)


def _scs_body(rcv_hbm, ef_hbm, zeros_hbm, acc_hbm,
              idx_r, ef_v, acc_sh, sem_e, sem_ir, sem_sc):
    cid = lax.axis_index("c")
    sid = lax.axis_index("s")
    base = (cid * NS + sid) * EPT
    arow = pl.multiple_of(sid * RPT, 8)

    def issue_idx(i, p):
        off = pl.multiple_of(base + i * CH, 16)
        pltpu.async_copy(rcv_hbm.at[pl.ds(off, CH)], idx_r.at[p], sem_ir.at[p])

    def wait_idx(p):
        pltpu.make_async_copy(rcv_hbm.at[pl.ds(0, CH)], idx_r.at[p],
                              sem_ir.at[p]).wait()

    def issue_ef(i, b):
        off = pl.multiple_of(base + i * CH, 16)
        pltpu.async_copy(ef_hbm.at[pl.ds(off, CH)], ef_v.at[b], sem_e.at[b])

    def wait_ef(b):
        pltpu.make_async_copy(ef_hbm.at[pl.ds(0, CH)], ef_v.at[b],
                              sem_e.at[b]).wait()

    def wait_sc(b):
        pltpu.make_async_copy(ef_hbm.at[pl.ds(0, CH)], ef_v.at[b],
                              sem_sc.at[b]).wait()

    for k in range(6):
        issue_idx(k, k)
    for k in range(3):
        issue_ef(k, k)
    pltpu.sync_copy(zeros_hbm.at[pl.ds(arow, RPT)],
                    acc_sh.at[pl.ds(arow, RPT)])
    plsc.subcore_barrier()

    def step(s, carry):
        for b in range(4):
            i = s * 4 + b
            b3 = (b + 3) % 4
            p = jnp.bitwise_and(i, 7)
            p6 = jnp.bitwise_and(i + 6, 7)

            wait_ef(b)
            wait_idx(p)
            pltpu.async_copy(ef_v.at[b], acc_sh.at[idx_r.at[p]],
                             sem_sc.at[b], add=True)

            @pl.when(i + 6 < NCH)
            def _():
                issue_idx(i + 6, p6)

            @pl.when(i + 3 < NCH)
            def _():
                @pl.when(i >= 1)
                def _():
                    wait_sc(b3)
                issue_ef(i + 3, b3)
        return carry

    lax.fori_loop(0, NCH // 4, step, 0)

    last = NCH - 1
    wait_ef(0)
    wait_idx(last % 8)
    pltpu.sync_copy(ef_v.at[0], acc_sh.at[idx_r.at[last % 8]], add=True)
    wait_sc(1)
    wait_sc(2)
    wait_sc(3)

    plsc.subcore_barrier()
    pltpu.sync_copy(acc_sh.at[pl.ds(arow, RPT)],
                    acc_hbm.at[cid, pl.ds(arow, RPT)])


_sc_scatter = pl.kernel(
    _scs_body,
    mesh=plsc.VectorSubcoreMesh(core_axis_name="c", subcore_axis_name="s"),
    out_type=[jax.ShapeDtypeStruct((NC, ACC_R, H), jnp.float32)],
    scratch_types=[
        pltpu.VMEM((8, CH), jnp.int32),
        pltpu.VMEM((4, CH, H), jnp.float32),
        pltpu.VMEM_SHARED((ACC_R, H), jnp.float32),
        pltpu.SemaphoreType.DMA((4,)),
        pltpu.SemaphoreType.DMA((8,)),
        pltpu.SemaphoreType.DMA((4,)),
    ],
)


def _pre_body(nf_ref, wa_ref, wb_ref, a_ref, b_ref):
    x = nf_ref[...]
    a_ref[...] = jnp.dot(x, wa_ref[...], preferred_element_type=jnp.float32)
    b_ref[...] = jnp.dot(x, wb_ref[...], preferred_element_type=jnp.float32)


def _layer_norm(x, g, b):
    m = jnp.mean(x, axis=-1, keepdims=True)
    v = jnp.mean((x - m) * (x - m), axis=-1, keepdims=True)
    return (x - m) * lax.rsqrt(v + 1e-5) * g + b


def _edge_body(g_ref, ef_ref, w1_ref, b1_ref, w2_ref, b2_ref, ge_ref, be_ref,
               out_ref):
    gi = g_ref[...]
    gp = jnp.concatenate(
        [lax.bitcast_convert_type(gi << 16, jnp.float32),
         lax.bitcast_convert_type(gi & jnp.int32(-65536), jnp.float32)],
        axis=1)
    ef = ef_ref[...]
    h = (gp
         + jnp.dot(ef, w1_ref[...], preferred_element_type=jnp.float32))
    h = jnp.maximum(h + b1_ref[...], 0.0)
    eu = jnp.dot(h, w2_ref[...], preferred_element_type=jnp.float32) + b2_ref[...]
    out_ref[...] = ef + _layer_norm(eu, ge_ref[...], be_ref[...])


def _node_body(nf_ref, acc_ref, wa_ref, wb_ref, b1_ref, w2_ref, b2_ref,
               gn_ref, bn_ref, out_ref):
    nf = nf_ref[...]
    acc = (acc_ref[0] + acc_ref[1])[:N_NODES]
    h = (jnp.dot(nf, wa_ref[...], preferred_element_type=jnp.float32)
         + jnp.dot(acc, wb_ref[...], preferred_element_type=jnp.float32))
    h = jnp.maximum(h + b1_ref[...], 0.0)
    nu = jnp.dot(h, w2_ref[...], preferred_element_type=jnp.float32) + b2_ref[...]
    out_ref[...] = nf + _layer_norm(nu, gn_ref[...], bn_ref[...])


RE = 4000


def kernel(edge_idx, node_feats, edge_feats, W1e, b1e, W2e, b2e, ge, be,
           W1n, b1n, W2n, b2n, gn, bn):
    senders = edge_idx[:, 0]
    receivers = edge_idx[:, 1]
    zeros = jnp.zeros((ACC_R, H), jnp.float32)

    tab_a, tab_b = pl.pallas_call(
        _pre_body,
        out_shape=[jax.ShapeDtypeStruct((N_NODES, H), jnp.float32)] * 2,
    )(node_feats, W1e[:H], W1e[H:2 * H])

    (g,) = _sc_gather(senders, receivers, tab_a, tab_b)
    (acc2,) = _sc_scatter(receivers, edge_feats, zeros)

    row = lambda v: v.reshape(1, H)
    edge_out = pl.pallas_call(
        _edge_body,
        grid=(N_EDGES // RE,),
        in_specs=[
            pl.BlockSpec((RE, HP), lambda i: (i, 0)),
            pl.BlockSpec((RE, H), lambda i: (i, 0)),
            pl.BlockSpec((H, H), lambda i: (0, 0)),
            pl.BlockSpec((1, H), lambda i: (0, 0)),
            pl.BlockSpec((H, H), lambda i: (0, 0)),
            pl.BlockSpec((1, H), lambda i: (0, 0)),
            pl.BlockSpec((1, H), lambda i: (0, 0)),
            pl.BlockSpec((1, H), lambda i: (0, 0)),
        ],
        out_specs=pl.BlockSpec((RE, H), lambda i: (i, 0)),
        out_shape=jax.ShapeDtypeStruct((N_EDGES, H), jnp.float32),
    )(g, edge_feats, W1e[2 * H:], row(b1e), W2e, row(b2e), row(ge), row(be))

    node_out = pl.pallas_call(
        _node_body,
        out_shape=jax.ShapeDtypeStruct((N_NODES, H), jnp.float32),
    )(node_feats, acc2, W1n[:H], W1n[H:], row(b1n), W2n, row(b2n),
      row(gn), row(bn))

    return (node_out, edge_out)


# speedup vs baseline: 4.4010x; 1.0069x over previous
import jax
import jax.numpy as jnp
from jax import lax
from jax.experimental import pallas as pl
from jax.experimental.pallas import tpu as pltpu
from jax.experimental.pallas import tpu_sc as plsc

H = 128
HP = H // 2
N_NODES = 10000
N_EDGES = 320000

NC = 2
NS = 16
NW = NC * NS
EPT = N_EDGES // NW
SUB = 80
CH = 160
NCH = EPT // CH
TAIL = EPT - NCH * CH
ACC_R = 10240
RPT = ACC_R // NS


def _scg_body(snd_hbm, rcv_hbm, ta_hbm, tb_hbm, g_hbm,
              idx_s, idx_r, rows_a, rows_b, rows_g, sem_a, sem_b, sem_w,
              sem_is, sem_ir):
    cid = lax.axis_index("c")
    sid = lax.axis_index("s")
    base = (cid * NS + sid) * EPT

    def issue_idx(i, p):
        off = pl.multiple_of(base + i * CH, 16)
        for k in range(2):
            pltpu.async_copy(snd_hbm.at[pl.ds(off + k * SUB, SUB)],
                             idx_s.at[p, k], sem_is.at[p])
            pltpu.async_copy(rcv_hbm.at[pl.ds(off + k * SUB, SUB)],
                             idx_r.at[p, k], sem_ir.at[p])

    def wait_idx(p):
        for k in range(2):
            pltpu.make_async_copy(snd_hbm.at[pl.ds(0, SUB)], idx_s.at[p, k],
                                  sem_is.at[p]).wait()
            pltpu.make_async_copy(rcv_hbm.at[pl.ds(0, SUB)], idx_r.at[p, k],
                                  sem_ir.at[p]).wait()

    def issue(i, p, b):
        for k in range(2):
            sl = pl.ds(k * SUB, SUB)
            pltpu.async_copy(ta_hbm.at[idx_s.at[p, k]], rows_a.at[b, sl],
                             sem_a.at[b])
            pltpu.async_copy(tb_hbm.at[idx_r.at[p, k]], rows_b.at[b, sl],
                             sem_b.at[b])

    def wait_in(b):
        for k in range(2):
            sl = pl.ds(k * SUB, SUB)
            pltpu.make_async_copy(ta_hbm.at[pl.ds(0, SUB)], rows_a.at[b, sl],
                                  sem_a.at[b]).wait()
            pltpu.make_async_copy(tb_hbm.at[pl.ds(0, SUB)], rows_b.at[b, sl],
                                  sem_b.at[b]).wait()

    def wait_w(b):
        pltpu.make_async_copy(rows_g.at[b], g_hbm.at[pl.ds(0, CH)],
                              sem_w.at[b]).wait()

    half = jnp.int32(0x8000)
    mask = jnp.int32(-65536)

    def add_rows(b, n):
        def add_row(r, c):
            for j in range(HP // 16):
                lo = (rows_a[b, r, pl.ds(j * 16, 16)]
                      + rows_b[b, r, pl.ds(j * 16, 16)])
                hi = (rows_a[b, r, pl.ds(j * 16 + 64, 16)]
                      + rows_b[b, r, pl.ds(j * 16 + 64, 16)])
                lo_i = lax.shift_right_logical(
                    lax.bitcast_convert_type(lo, jnp.int32) + half, 16)
                hi_i = (lax.bitcast_convert_type(hi, jnp.int32) + half) & mask
                rows_g[b, r, pl.ds(j * 16, 16)] = lo_i | hi_i
            return c

        lax.fori_loop(0, n, add_row, 0)

    issue_idx(0, 0)
    issue_idx(1, 1)
    wait_idx(0)
    issue(0, 0, 0)

    def step(s, carry):
        for b in range(2):
            i = s * 2 + b
            nb = 1 - b
            p1 = jnp.bitwise_and(i + 1, 3)
            p2 = jnp.bitwise_and(i + 2, 3)

            wait_in(b)

            @pl.when(i + 2 < NCH)
            def _():
                issue_idx(i + 2, p2)

            @pl.when(i + 1 < NCH)
            def _():
                wait_idx(p1)

                @pl.when(i >= 1)
                def _():
                    wait_w(nb)
                issue(i + 1, p1, nb)

            add_rows(b, CH)
            off = pl.multiple_of(base + i * CH, 16)
            pltpu.async_copy(rows_g.at[b], g_hbm.at[pl.ds(off, CH)],
                             sem_w.at[b])
        return carry

    lax.fori_loop(0, NCH // 2, step, 0)

    toff = pl.multiple_of(base + NCH * CH, 16)
    wait_w(0)
    pltpu.sync_copy(snd_hbm.at[pl.ds(toff, SUB)], idx_s.at[0, 0])
    pltpu.sync_copy(rcv_hbm.at[pl.ds(toff, SUB)], idx_r.at[0, 0])
    tsl = pl.ds(0, SUB)
    pltpu.async_copy(ta_hbm.at[idx_s.at[0, 0]], rows_a.at[0, tsl],
                     sem_a.at[0])
    pltpu.async_copy(tb_hbm.at[idx_r.at[0, 0]], rows_b.at[0, tsl],
                     sem_b.at[0])
    pltpu.make_async_copy(ta_hbm.at[pl.ds(0, SUB)], rows_a.at[0, tsl],
                          sem_a.at[0]).wait()
    pltpu.make_async_copy(tb_hbm.at[pl.ds(0, SUB)], rows_b.at[0, tsl],
                          sem_b.at[0]).wait()
    add_rows(0, TAIL)
    pltpu.sync_copy(rows_g.at[0, tsl], g_hbm.at[pl.ds(toff, SUB)])
    wait_w(1)


_sc_gather = pl.kernel(
    _scg_body,
    mesh=plsc.VectorSubcoreMesh(core_axis_name="c", subcore_axis_name="s"),
    out_type=[jax.ShapeDtypeStruct((N_EDGES, HP), jnp.int32)],
    scratch_types=[
        pltpu.VMEM((4, 2, SUB), jnp.int32),
        pltpu.VMEM((4, 2, SUB), jnp.int32),
        pltpu.VMEM((2, CH, H), jnp.float32),
        pltpu.VMEM((2, CH, H), jnp.float32),
        pltpu.VMEM((2, CH, HP), jnp.int32),
        pltpu.SemaphoreType.DMA((2,)),
        pltpu.SemaphoreType.DMA((2,)),
        pltpu.SemaphoreType.DMA((2,)),
        pltpu.SemaphoreType.DMA((4,)),
        pltpu.SemaphoreType.DMA((4,)),
    ],
)


def _scs_body(rcv_hbm, ef_hbm, zeros_hbm, acc_hbm,
              idx_r, ef_v, acc_sh, sem_e, sem_ir, sem_sc):
    cid = lax.axis_index("c")
    sid = lax.axis_index("s")
    base = (cid * NS + sid) * EPT
    arow = pl.multiple_of(sid * RPT, 8)

    def issue_idx(i, p):
        off = pl.multiple_of(base + i * CH, 16)
        for k in range(2):
            pltpu.async_copy(rcv_hbm.at[pl.ds(off + k * SUB, SUB)],
                             idx_r.at[p, k], sem_ir.at[p])

    def wait_idx(p):
        for k in range(2):
            pltpu.make_async_copy(rcv_hbm.at[pl.ds(0, SUB)], idx_r.at[p, k],
                                  sem_ir.at[p]).wait()

    def issue_ef(i, b):
        off = pl.multiple_of(base + i * CH, 16)
        pltpu.async_copy(ef_hbm.at[pl.ds(off, CH)], ef_v.at[b], sem_e.at[b])

    def wait_ef(b):
        pltpu.make_async_copy(ef_hbm.at[pl.ds(0, CH)], ef_v.at[b],
                              sem_e.at[b]).wait()

    def wait_sc(b):
        for k in range(2):
            pltpu.make_async_copy(ef_hbm.at[pl.ds(0, SUB)],
                                  ef_v.at[b, pl.ds(k * SUB, SUB)],
                                  sem_sc.at[b]).wait()

    issue_idx(0, 0)
    issue_idx(1, 1)
    issue_ef(0, 0)
    pltpu.sync_copy(zeros_hbm.at[pl.ds(arow, RPT)],
                    acc_sh.at[pl.ds(arow, RPT)])
    plsc.subcore_barrier()

    def step(s, carry):
        for b in range(2):
            i = s * 2 + b
            nb = 1 - b
            p = jnp.bitwise_and(i, 3)
            p2 = jnp.bitwise_and(i + 2, 3)

            wait_ef(b)
            wait_idx(p)
            for k in range(2):
                pltpu.async_copy(ef_v.at[b, pl.ds(k * SUB, SUB)],
                                 acc_sh.at[idx_r.at[p, k]],
                                 sem_sc.at[b], add=True)

            @pl.when(i + 2 < NCH)
            def _():
                issue_idx(i + 2, p2)

            @pl.when(i + 1 < NCH)
            def _():
                @pl.when(i >= 1)
                def _():
                    wait_sc(nb)
                issue_ef(i + 1, nb)
        return carry

    lax.fori_loop(0, NCH // 2, step, 0)

    toff = pl.multiple_of(base + NCH * CH, 16)
    wait_sc(0)
    pltpu.sync_copy(rcv_hbm.at[pl.ds(toff, SUB)], idx_r.at[0, 0])
    pltpu.sync_copy(ef_hbm.at[pl.ds(toff, SUB)], ef_v.at[0, pl.ds(0, SUB)])
    pltpu.sync_copy(ef_v.at[0, pl.ds(0, SUB)], acc_sh.at[idx_r.at[0, 0]],
                    add=True)
    wait_sc(1)

    plsc.subcore_barrier()
    pltpu.sync_copy(acc_sh.at[pl.ds(arow, RPT)],
                    acc_hbm.at[cid, pl.ds(arow, RPT)])


_sc_scatter = pl.kernel(
    _scs_body,
    mesh=plsc.VectorSubcoreMesh(core_axis_name="c", subcore_axis_name="s"),
    out_type=[jax.ShapeDtypeStruct((NC, ACC_R, H), jnp.float32)],
    scratch_types=[
        pltpu.VMEM((4, 2, SUB), jnp.int32),
        pltpu.VMEM((2, CH, H), jnp.float32),
        pltpu.VMEM_SHARED((ACC_R, H), jnp.float32),
        pltpu.SemaphoreType.DMA((2,)),
        pltpu.SemaphoreType.DMA((4,)),
        pltpu.SemaphoreType.DMA((2,)),
    ],
)


def _pre_body(nf_ref, wa_ref, wb_ref, a_ref, b_ref):
    x = nf_ref[...]
    a_ref[...] = jnp.dot(x, wa_ref[...], preferred_element_type=jnp.float32)
    b_ref[...] = jnp.dot(x, wb_ref[...], preferred_element_type=jnp.float32)


def _layer_norm(x, g, b):
    m = jnp.mean(x, axis=-1, keepdims=True)
    v = jnp.mean((x - m) * (x - m), axis=-1, keepdims=True)
    return (x - m) * lax.rsqrt(v + 1e-5) * g + b


def _edge_body(g_ref, ef_ref, w1_ref, b1_ref, w2_ref, b2_ref, ge_ref, be_ref,
               out_ref):
    gi = g_ref[...]
    gp = jnp.concatenate(
        [lax.bitcast_convert_type(gi << 16, jnp.float32),
         lax.bitcast_convert_type(gi & jnp.int32(-65536), jnp.float32)],
        axis=1)
    ef = ef_ref[...]
    h = (gp
         + jnp.dot(ef, w1_ref[...], preferred_element_type=jnp.float32))
    h = jnp.maximum(h + b1_ref[...], 0.0)
    eu = jnp.dot(h, w2_ref[...], preferred_element_type=jnp.float32) + b2_ref[...]
    out_ref[...] = ef + _layer_norm(eu, ge_ref[...], be_ref[...])


def _node_body(nf_ref, acc_ref, wa_ref, wb_ref, b1_ref, w2_ref, b2_ref,
               gn_ref, bn_ref, out_ref):
    nf = nf_ref[...]
    acc = (acc_ref[0] + acc_ref[1])[:N_NODES]
    h = (jnp.dot(nf, wa_ref[...], preferred_element_type=jnp.float32)
         + jnp.dot(acc, wb_ref[...], preferred_element_type=jnp.float32))
    h = jnp.maximum(h + b1_ref[...], 0.0)
    nu = jnp.dot(h, w2_ref[...], preferred_element_type=jnp.float32) + b2_ref[...]
    out_ref[...] = nf + _layer_norm(nu, gn_ref[...], bn_ref[...])


RE = 4000


def kernel(edge_idx, node_feats, edge_feats, W1e, b1e, W2e, b2e, ge, be,
           W1n, b1n, W2n, b2n, gn, bn):
    senders = edge_idx[:, 0]
    receivers = edge_idx[:, 1]
    zeros = jnp.zeros((ACC_R, H), jnp.float32)

    tab_a, tab_b = pl.pallas_call(
        _pre_body,
        out_shape=[jax.ShapeDtypeStruct((N_NODES, H), jnp.float32)] * 2,
    )(node_feats, W1e[:H], W1e[H:2 * H])

    (g,) = _sc_gather(senders, receivers, tab_a, tab_b)
    (acc2,) = _sc_scatter(receivers, edge_feats, zeros)

    row = lambda v: v.reshape(1, H)
    edge_out = pl.pallas_call(
        _edge_body,
        grid=(N_EDGES // RE,),
        in_specs=[
            pl.BlockSpec((RE, HP), lambda i: (i, 0)),
            pl.BlockSpec((RE, H), lambda i: (i, 0)),
            pl.BlockSpec((H, H), lambda i: (0, 0)),
            pl.BlockSpec((1, H), lambda i: (0, 0)),
            pl.BlockSpec((H, H), lambda i: (0, 0)),
            pl.BlockSpec((1, H), lambda i: (0, 0)),
            pl.BlockSpec((1, H), lambda i: (0, 0)),
            pl.BlockSpec((1, H), lambda i: (0, 0)),
        ],
        out_specs=pl.BlockSpec((RE, H), lambda i: (i, 0)),
        out_shape=jax.ShapeDtypeStruct((N_EDGES, H), jnp.float32),
    )(g, edge_feats, W1e[2 * H:], row(b1e), W2e, row(b2e), row(ge), row(be))

    node_out = pl.pallas_call(
        _node_body,
        out_shape=jax.ShapeDtypeStruct((N_NODES, H), jnp.float32),
    )(node_feats, acc2, W1n[:H], W1n[H:], row(b1n), W2n, row(b2n),
      row(gn), row(bn))

    return (node_out, edge_out)
